# baseline (speedup 1.0000x reference)
import jax
import jax.numpy as jnp
from jax.experimental import pallas as pl
from jax.experimental.pallas import tpu as pltpu

_TS = 2048
_G = 64


def _router_body(x_ref, gt_ref, w_ref, b_ref, sl_ref, out_ref, sem, sem2, gsem):
    bi = pl.program_id(0)
    i = pl.program_id(1)

    half = _TS // 2
    cp = pltpu.make_async_copy(
        x_ref.at[0, pl.ds(0, half), :],
        out_ref.at[bi, pl.ds(_G + i * _TS, half), :],
        sem,
    )
    cp.start()
    cp2 = pltpu.make_async_copy(
        x_ref.at[0, pl.ds(half, half), :],
        out_ref.at[bi, pl.ds(_G + i * _TS + half, half), :],
        sem2,
    )
    cp2.start()

    @pl.when(i == 0)
    def _():
        gcp = pltpu.make_async_copy(gt_ref, out_ref.at[bi, pl.ds(0, _G), :], gsem)
        gcp.start()
        gcp.wait()

    xb = x_ref[0]
    r = jnp.dot(xb, w_ref[...], preferred_element_type=jnp.float32)
    r = r + b_ref[...]
    lane = jax.lax.broadcasted_iota(jnp.int32, r.shape, 1)
    sl_ref[0] = jnp.where(lane == 0, jax.nn.sigmoid(r), r)

    cp.wait()
    cp2.wait()


def kernel(x, global_tokens, anchor_w, anchor_b, seg_w, seg_b):
    b, s, h = x.shape
    g = global_tokens.shape[0]
    n = s // _TS

    w = jnp.concatenate([anchor_w, seg_w], axis=1)
    bias = jnp.stack([anchor_b[0], seg_b[0]]).reshape(1, 2)

    sl, out = pl.pallas_call(
        _router_body,
        grid=(b, n),
        in_specs=[
            pl.BlockSpec((1, _TS, h), lambda i, j: (i, j, 0)),
            pl.BlockSpec((g, h), lambda i, j: (0, 0)),
            pl.BlockSpec((h, 2), lambda i, j: (0, 0)),
            pl.BlockSpec((1, 2), lambda i, j: (0, 0)),
        ],
        out_specs=[
            pl.BlockSpec((1, _TS, 2), lambda i, j: (i, j, 0)),
            pl.BlockSpec(memory_space=pltpu.HBM),
        ],
        out_shape=[
            jax.ShapeDtypeStruct((b, s, 2), jnp.float32),
            jax.ShapeDtypeStruct((b, g + s, h), jnp.float32),
        ],
        scratch_shapes=[pltpu.SemaphoreType.DMA, pltpu.SemaphoreType.DMA, pltpu.SemaphoreType.DMA],
    )(x, global_tokens, w, bias)

    anchor_scores = sl[:, :, 0]
    segment_logits = sl[:, :, 1]
    mask_row = jnp.arange(g + s, dtype=jnp.int32) < g
    global_mask = jnp.broadcast_to(mask_row[None, :], (b, g + s))
    return (out, global_mask, anchor_scores, segment_logits)


# speedup vs baseline: 1.1125x; 1.1125x over previous
import functools

import jax
import jax.numpy as jnp
from jax.experimental import pallas as pl
from jax.experimental.pallas import tpu as pltpu

_TS = 2048
_G = 64
_NBUF = 5


def _read(x_ref, buf, rsem, t, n_per_b, slot):
    bt = t // n_per_b
    it = t % n_per_b
    return pltpu.make_async_copy(
        x_ref.at[bt, pl.ds(it * _TS, _TS), :], buf.at[slot], rsem.at[slot]
    )


def _write(out_ref, buf, wsem, t, n_per_b, slot):
    bt = t // n_per_b
    it = t % n_per_b
    return pltpu.make_async_copy(
        buf.at[slot], out_ref.at[bt, pl.ds(_G + it * _TS, _TS), :], wsem.at[slot]
    )


def _router_body(x_ref, gt_ref, w_ref, b_ref, sl_ref, out_ref,
                 buf, rsem, wsem, gsem, *, n_per_b, n_total, n_batch):
    bi = pl.program_id(0)
    i = pl.program_id(1)
    g = bi * n_per_b + i

    @pl.when(g == 0)
    def _():
        for t in range(min(_NBUF - 1, n_total)):
            _read(x_ref, buf, rsem, t, n_per_b, t).start()
        for bb in range(n_batch):
            pltpu.make_async_copy(
                gt_ref, out_ref.at[bb, pl.ds(0, _G), :], gsem
            ).start()
        for bb in range(n_batch):
            pltpu.make_async_copy(
                gt_ref, out_ref.at[0, pl.ds(0, _G), :], gsem
            ).wait()

    t_pre = g + _NBUF - 1
    s_pre = t_pre % _NBUF

    @pl.when((g >= 1) & (t_pre <= n_total - 1))
    def _():
        _write(out_ref, buf, wsem, g - 1, n_per_b, s_pre).wait()

    @pl.when(t_pre <= n_total - 1)
    def _():
        _read(x_ref, buf, rsem, t_pre, n_per_b, s_pre).start()

    s = g % _NBUF
    _read(x_ref, buf, rsem, g, n_per_b, s).wait()

    xb = buf[s]
    r = jnp.dot(xb, w_ref[...], preferred_element_type=jnp.float32)
    r = r + b_ref[...]
    lane = jax.lax.broadcasted_iota(jnp.int32, r.shape, 1)
    sl_ref[0] = jnp.where(lane == 0, jax.nn.sigmoid(r), r)

    _write(out_ref, buf, wsem, g, n_per_b, s).start()

    @pl.when(g == n_total - 1)
    def _():
        for d in range(min(_NBUF, n_total)):
            t = n_total - 1 - d
            _write(out_ref, buf, wsem, t, n_per_b, t % _NBUF).wait()


def kernel(x, global_tokens, anchor_w, anchor_b, seg_w, seg_b):
    b, s, h = x.shape
    g = global_tokens.shape[0]
    n_per_b = s // _TS
    n_total = b * n_per_b

    w = jnp.concatenate([anchor_w, seg_w], axis=1)
    bias = jnp.stack([anchor_b[0], seg_b[0]]).reshape(1, 2)

    body = functools.partial(
        _router_body, n_per_b=n_per_b, n_total=n_total, n_batch=b
    )

    sl, out = pl.pallas_call(
        body,
        grid=(b, n_per_b),
        in_specs=[
            pl.BlockSpec(memory_space=pltpu.HBM),
            pl.BlockSpec((g, h), lambda i, j: (0, 0)),
            pl.BlockSpec((h, 2), lambda i, j: (0, 0)),
            pl.BlockSpec((1, 2), lambda i, j: (0, 0)),
        ],
        out_specs=[
            pl.BlockSpec((1, _TS, 2), lambda i, j: (i, j, 0)),
            pl.BlockSpec(memory_space=pltpu.HBM),
        ],
        out_shape=[
            jax.ShapeDtypeStruct((b, s, 2), jnp.float32),
            jax.ShapeDtypeStruct((b, g + s, h), jnp.float32),
        ],
        scratch_shapes=[
            pltpu.VMEM((_NBUF, _TS, h), jnp.float32),
            pltpu.SemaphoreType.DMA((_NBUF,)),
            pltpu.SemaphoreType.DMA((_NBUF,)),
            pltpu.SemaphoreType.DMA,
        ],
    )(x, global_tokens, w, bias)

    anchor_scores = sl[:, :, 0]
    segment_logits = sl[:, :, 1]
    mask_row = jnp.arange(g + s, dtype=jnp.int32) < g
    global_mask = jnp.broadcast_to(mask_row[None, :], (b, g + s))
    return (out, global_mask, anchor_scores, segment_logits)
